# baseline (speedup 1.0000x reference)
import functools

import jax
import jax.numpy as jnp
from jax import lax
from jax.experimental import pallas as pl
from jax.experimental.pallas import tpu as pltpu
from jax.experimental.pallas import tpu_sc as plsc

_N = 50000
_D = 256
_EPS = 1e-12
_RB = 80
_BW = _RB * _D
_NBLK = _N // _RB
_NW = 32
_MAXITER = -(-_NBLK // _NW)


def _rsqrt_nr(s):
    i = lax.bitcast_convert_type(s, jnp.int32)
    i = jnp.int32(0x5F3759DF) - lax.shift_right_logical(i, 1)
    r = lax.bitcast_convert_type(i, jnp.float32)
    for _ in range(3):
        r = r * (1.5 - 0.5 * s * r * r)
    return r


def _pair_contrib(sp, sq, d):
    rp = jnp.where(sp >= _EPS * _EPS, _rsqrt_nr(sp), 1.0 / _EPS)
    rq = jnp.where(sq >= _EPS * _EPS, _rsqrt_nr(sq), 1.0 / _EPS)
    return sp * rp * rp + sq * rq * rq - 2.0 * d * rp * rq


def _diag_indices():
    lanes = lax.iota(jnp.int32, 16)
    out = []
    for k in range(16):
        col = jnp.where(lanes < 16 - k, lanes + k, lanes + k - 16)
        out.append(lanes * _D + col)
    return out


def _group_contrib(bp, bq, idxs, gbase):
    zeros = jnp.zeros((16,), jnp.float32)

    def sbody(s, carry):
        spv, sqv, dv = carry
        base = gbase + s * 16
        for k in range(16):
            off = idxs[k] + base
            vp = plsc.load_gather(bp, [off])
            vq = plsc.load_gather(bq, [off])
            spv = spv + vp * vp
            sqv = sqv + vq * vq
            dv = dv + vp * vq
        return spv, sqv, dv

    spv, sqv, dv = lax.fori_loop(0, _D // 16, sbody, (zeros,) * 3)
    return _pair_contrib(spv, sqv, dv)


def _make_sc_call():
    mesh = plsc.VectorSubcoreMesh(core_axis_name="c", subcore_axis_name="s")

    @functools.partial(
        pl.kernel,
        mesh=mesh,
        compiler_params=pltpu.CompilerParams(needs_layout_passes=False),
        out_type=jax.ShapeDtypeStruct((_NW * 16,), jnp.float32),
        scratch_types=[
            pltpu.VMEM((_BW,), jnp.float32),
            pltpu.VMEM((_BW,), jnp.float32),
            pltpu.VMEM((_BW,), jnp.float32),
            pltpu.VMEM((_BW,), jnp.float32),
            pltpu.VMEM((16,), jnp.float32),
            pltpu.SemaphoreType.DMA,
            pltpu.SemaphoreType.DMA,
        ],
    )
    def sc_call(p1h, q2h, p2h, q1h, outh,
                ap, aq, bp, bq, accv, sem_a, sem_b):
        c = lax.axis_index("c")
        s = lax.axis_index("s")
        wid = s * 2 + c
        accv[...] = jnp.zeros((16,), jnp.float32)
        idxs = _diag_indices()
        slots = ((ap, aq, sem_a), (bp, bq, sem_b))

        def phase(ph, qh):
            hbm = (ph, qh)

            def issue(i, slot):
                blk = wid + i * _NW

                @pl.when(blk < _NBLK)
                def _():
                    base = blk * _BW
                    for src, dst in zip(hbm, slot[:2]):
                        pltpu.async_copy(
                            src.at[pl.ds(base, _BW)], dst, slot[2])

            def drain_compute(i, slot):
                blk = wid + i * _NW

                @pl.when(blk < _NBLK)
                def _():
                    base = blk * _BW
                    for src, dst in zip(hbm, slot[:2]):
                        pltpu.make_async_copy(
                            src.at[pl.ds(base, _BW)], dst, slot[2]).wait()
                    for g in range(_RB // 16):
                        accv[...] = accv[...] + _group_contrib(
                            slot[0], slot[1], idxs, g * 16 * _D)

            issue(0, slots[0])

            def pair_body(i2, _):
                i = i2 * 2
                issue(i + 1, slots[1])
                drain_compute(i, slots[0])
                issue(i + 2, slots[0])
                drain_compute(i + 1, slots[1])
                return 0

            lax.fori_loop(0, _MAXITER // 2, pair_body, 0)

        phase(p1h, q2h)
        phase(p2h, q1h)
        pltpu.sync_copy(accv, outh.at[pl.ds(wid * 16, 16)])

    return sc_call


_sc_call = _make_sc_call()


def kernel(pred1, proj2, pred2, proj1):
    partials = _sc_call(
        pred1.reshape(-1), proj2.reshape(-1),
        pred2.reshape(-1), proj1.reshape(-1))
    return 0.5 * jnp.sum(partials)


# speedup vs baseline: 1.1039x; 1.1039x over previous
import functools

import jax
import jax.numpy as jnp
from jax import lax
from jax.experimental import pallas as pl
from jax.experimental.pallas import tpu as pltpu
from jax.experimental.pallas import tpu_sc as plsc

_N = 50000
_D = 256
_EPS = 1e-12
_RB = 80
_BW = _RB * _D
_NW = 32

_N_SC = 16000
_TC_B = 1000
_N_TC = _N - _N_SC
_NBLK = _N_SC // _RB
_MAXITER = 2 * (-(-(-(-_NBLK // _NW)) // 2))


def _rsqrt_nr(s):
    i = lax.bitcast_convert_type(s, jnp.int32)
    i = jnp.int32(0x5F3759DF) - lax.shift_right_logical(i, 1)
    r = lax.bitcast_convert_type(i, jnp.float32)
    for _ in range(3):
        r = r * (1.5 - 0.5 * s * r * r)
    return r


def _pair_contrib(sp, sq, d):
    rp = jnp.where(sp >= _EPS * _EPS, _rsqrt_nr(sp), 1.0 / _EPS)
    rq = jnp.where(sq >= _EPS * _EPS, _rsqrt_nr(sq), 1.0 / _EPS)
    return sp * rp * rp + sq * rq * rq - 2.0 * d * rp * rq


def _diag_indices():
    lanes = lax.iota(jnp.int32, 16)
    out = []
    for k in range(16):
        col = jnp.where(lanes < 16 - k, lanes + k, lanes + k - 16)
        out.append(lanes * _D + col)
    return out


def _group_contrib(bp, bq, idxs, gbase):
    zeros = jnp.zeros((16,), jnp.float32)

    def sbody(s, carry):
        spv, sqv, dv = carry
        base = gbase + s * 16
        for k in range(16):
            off = idxs[k] + base
            vp = plsc.load_gather(bp, [off])
            vq = plsc.load_gather(bq, [off])
            spv = spv + vp * vp
            sqv = sqv + vq * vq
            dv = dv + vp * vq
        return spv, sqv, dv

    spv, sqv, dv = lax.fori_loop(0, _D // 16, sbody, (zeros,) * 3)
    return _pair_contrib(spv, sqv, dv)


def _make_sc_call():
    mesh = plsc.VectorSubcoreMesh(core_axis_name="c", subcore_axis_name="s")

    @functools.partial(
        pl.kernel,
        mesh=mesh,
        compiler_params=pltpu.CompilerParams(needs_layout_passes=False),
        out_type=jax.ShapeDtypeStruct((_NW * 16,), jnp.float32),
        scratch_types=[
            pltpu.VMEM((_BW,), jnp.float32),
            pltpu.VMEM((_BW,), jnp.float32),
            pltpu.VMEM((_BW,), jnp.float32),
            pltpu.VMEM((_BW,), jnp.float32),
            pltpu.VMEM((16,), jnp.float32),
            pltpu.SemaphoreType.DMA,
            pltpu.SemaphoreType.DMA,
        ],
    )
    def sc_call(p1h, q2h, p2h, q1h, outh,
                ap, aq, bp, bq, accv, sem_a, sem_b):
        c = lax.axis_index("c")
        s = lax.axis_index("s")
        wid = s * 2 + c
        accv[...] = jnp.zeros((16,), jnp.float32)
        idxs = _diag_indices()
        slots = ((ap, aq, sem_a), (bp, bq, sem_b))

        def phase(ph, qh):
            hbm = (ph, qh)

            def issue(i, slot):
                blk = wid + i * _NW

                @pl.when(blk < _NBLK)
                def _():
                    base = blk * _BW
                    for src, dst in zip(hbm, slot[:2]):
                        pltpu.async_copy(
                            src.at[pl.ds(base, _BW)], dst, slot[2])

            def drain_compute(i, slot):
                blk = wid + i * _NW

                @pl.when(blk < _NBLK)
                def _():
                    base = blk * _BW
                    for src, dst in zip(hbm, slot[:2]):
                        pltpu.make_async_copy(
                            src.at[pl.ds(base, _BW)], dst, slot[2]).wait()
                    for g in range(_RB // 16):
                        accv[...] = accv[...] + _group_contrib(
                            slot[0], slot[1], idxs, g * 16 * _D)

            issue(0, slots[0])

            def pair_body(i2, _):
                i = i2 * 2
                issue(i + 1, slots[1])
                drain_compute(i, slots[0])
                issue(i + 2, slots[0])
                drain_compute(i + 1, slots[1])
                return 0

            lax.fori_loop(0, _MAXITER // 2, pair_body, 0)

        phase(p1h, q2h)
        phase(p2h, q1h)
        pltpu.sync_copy(accv, outh.at[pl.ds(wid * 16, 16)])

    return sc_call


_sc_call = _make_sc_call()


def _tc_body(p1, q2, p2, q1, out):
    def pair(a, b):
        av = a[...]
        bv = b[...]
        sa = jnp.sum(av * av, axis=1, keepdims=True)
        sb = jnp.sum(bv * bv, axis=1, keepdims=True)
        d = jnp.sum(av * bv, axis=1, keepdims=True)
        na = jnp.maximum(jnp.sqrt(sa), _EPS)
        nb = jnp.maximum(jnp.sqrt(sb), _EPS)
        return sa / (na * na) + sb / (nb * nb) - 2.0 * d / (na * nb)

    c = pair(p1, q2) + pair(p2, q1)
    out[...] = jnp.full((8, 128), jnp.sum(c), jnp.float32)


_GRID_TC = _N_TC // _TC_B
_OFF_TC = _N_SC // _TC_B


def _tc_call(pred1, proj2, pred2, proj1):
    in_spec = pl.BlockSpec((_TC_B, _D), lambda i: (i + _OFF_TC, 0))
    return pl.pallas_call(
        _tc_body,
        grid=(_GRID_TC,),
        in_specs=[in_spec] * 4,
        out_specs=pl.BlockSpec((8, 128), lambda i: (i, 0)),
        out_shape=jax.ShapeDtypeStruct((_GRID_TC * 8, 128), jnp.float32),
    )(pred1, proj2, pred2, proj1)


def kernel(pred1, proj2, pred2, proj1):
    sc_partials = _sc_call(
        pred1.reshape(-1), proj2.reshape(-1),
        pred2.reshape(-1), proj1.reshape(-1))
    tc_partials = _tc_call(pred1, proj2, pred2, proj1)
    return 0.5 * (jnp.sum(sc_partials)
                  + jnp.sum(tc_partials) * (1.0 / (8.0 * 128.0)))


# speedup vs baseline: 1.4948x; 1.3541x over previous
import functools

import jax
import jax.numpy as jnp
from jax import lax
from jax.experimental import pallas as pl
from jax.experimental.pallas import tpu as pltpu
from jax.experimental.pallas import tpu_sc as plsc

_N = 50000
_D = 256
_EPS = 1e-12
_RB = 80
_BW = _RB * _D
_NW = 32

_N_SC = 16000
_TC_B = 1000
_N_TC = _N - _N_SC
_NBLK = _N_SC // _RB
_MAXITER = 2 * (-(-(-(-_NBLK // _NW)) // 2))


def _rsqrt_nr(s):
    i = lax.bitcast_convert_type(s, jnp.int32)
    i = jnp.int32(0x5F3759DF) - lax.shift_right_logical(i, 1)
    r = lax.bitcast_convert_type(i, jnp.float32)
    for _ in range(3):
        r = r * (1.5 - 0.5 * s * r * r)
    return r


def _pair_contrib(sp, sq, d):
    rp = jnp.where(sp >= _EPS * _EPS, _rsqrt_nr(sp), 1.0 / _EPS)
    rq = jnp.where(sq >= _EPS * _EPS, _rsqrt_nr(sq), 1.0 / _EPS)
    return sp * rp * rp + sq * rq * rq - 2.0 * d * rp * rq


def _diag_indices():
    lanes = lax.iota(jnp.int32, 16)
    out = []
    for k in range(16):
        col = jnp.where(lanes < 16 - k, lanes + k, lanes + k - 16)
        out.append(lanes * _D + col)
    return out


def _group_contrib(bp, bq, idxs, gbase):
    zeros = jnp.zeros((16,), jnp.float32)

    def sbody(s, carry):
        spv, sqv, dv = carry
        base = gbase + s * 16
        for k in range(16):
            off = idxs[k] + base
            vp = plsc.load_gather(bp, [off])
            vq = plsc.load_gather(bq, [off])
            spv = spv + vp * vp
            sqv = sqv + vq * vq
            dv = dv + vp * vq
        return spv, sqv, dv

    spv, sqv, dv = lax.fori_loop(0, _D // 16, sbody, (zeros,) * 3)
    return _pair_contrib(spv, sqv, dv)


def _make_sc_call():
    mesh = plsc.VectorSubcoreMesh(core_axis_name="c", subcore_axis_name="s")

    @functools.partial(
        pl.kernel,
        mesh=mesh,
        compiler_params=pltpu.CompilerParams(needs_layout_passes=False),
        out_type=jax.ShapeDtypeStruct((_NW * 16,), jnp.float32),
        scratch_types=[
            pltpu.VMEM((_BW,), jnp.float32),
            pltpu.VMEM((_BW,), jnp.float32),
            pltpu.VMEM((_BW,), jnp.float32),
            pltpu.VMEM((_BW,), jnp.float32),
            pltpu.VMEM((16,), jnp.float32),
            pltpu.SemaphoreType.DMA,
            pltpu.SemaphoreType.DMA,
        ],
    )
    def sc_call(p1h, q2h, p2h, q1h, outh,
                ap, aq, bp, bq, accv, sem_a, sem_b):
        c = lax.axis_index("c")
        s = lax.axis_index("s")
        wid = s * 2 + c
        accv[...] = jnp.zeros((16,), jnp.float32)
        idxs = _diag_indices()
        slots = ((ap, aq, sem_a), (bp, bq, sem_b))

        def phase(ph, qh):
            hbm = (ph, qh)

            def issue(i, slot):
                blk = wid + i * _NW

                @pl.when(blk < _NBLK)
                def _():
                    base = blk * _BW
                    for src, dst in zip(hbm, slot[:2]):
                        pltpu.async_copy(
                            src.at[pl.ds(base, _BW)], dst, slot[2])

            def drain_compute(i, slot):
                blk = wid + i * _NW

                @pl.when(blk < _NBLK)
                def _():
                    base = blk * _BW
                    for src, dst in zip(hbm, slot[:2]):
                        pltpu.make_async_copy(
                            src.at[pl.ds(base, _BW)], dst, slot[2]).wait()
                    for g in range(_RB // 16):
                        accv[...] = accv[...] + _group_contrib(
                            slot[0], slot[1], idxs, g * 16 * _D)

            issue(0, slots[0])

            def pair_body(i2, _):
                i = i2 * 2
                issue(i + 1, slots[1])
                drain_compute(i, slots[0])
                issue(i + 2, slots[0])
                drain_compute(i + 1, slots[1])
                return 0

            lax.fori_loop(0, _MAXITER // 2, pair_body, 0)

        phase(p1h, q2h)
        phase(p2h, q1h)
        pltpu.sync_copy(accv, outh.at[pl.ds(wid * 16, 16)])

    return sc_call


_sc_call = _make_sc_call()


def _tc_body(p1, q2, p2, q1, out):
    def pair(a, b):
        av = a[...]
        bv = b[...]
        sa = jnp.sum(av * av, axis=1, keepdims=True)
        sb = jnp.sum(bv * bv, axis=1, keepdims=True)
        d = jnp.sum(av * bv, axis=1, keepdims=True)
        na = jnp.maximum(jnp.sqrt(sa), _EPS)
        nb = jnp.maximum(jnp.sqrt(sb), _EPS)
        return sa / (na * na) + sb / (nb * nb) - 2.0 * d / (na * nb)

    c = pair(p1, q2) + pair(p2, q1)
    out[...] = jnp.full((8, 128), jnp.sum(c), jnp.float32)


_GRID_TC = _N_TC // _TC_B
_OFF_TC = _N_SC // _TC_B


def _tc_call(pred1, proj2, pred2, proj1):
    in_spec = pl.BlockSpec((_TC_B, _D), lambda i: (i + _OFF_TC, 0))
    return pl.pallas_call(
        _tc_body,
        grid=(_GRID_TC,),
        in_specs=[in_spec] * 4,
        out_specs=pl.BlockSpec((8, 128), lambda i: (i, 0)),
        out_shape=jax.ShapeDtypeStruct((_GRID_TC * 8, 128), jnp.float32),
    )(pred1, proj2, pred2, proj1)


def kernel(pred1, proj2, pred2, proj1):
    sc_partials = _sc_call(
        pred1[:_N_SC].reshape(-1), proj2[:_N_SC].reshape(-1),
        pred2[:_N_SC].reshape(-1), proj1[:_N_SC].reshape(-1))
    tc_partials = _tc_call(pred1, proj2, pred2, proj1)
    return 0.5 * (jnp.sum(sc_partials)
                  + jnp.sum(tc_partials) * (1.0 / (8.0 * 128.0)))


# speedup vs baseline: 1.6971x; 1.1353x over previous
import functools

import jax
import jax.numpy as jnp
from jax import lax
from jax.experimental import pallas as pl
from jax.experimental.pallas import tpu as pltpu
from jax.experimental.pallas import tpu_sc as plsc

_N = 50000
_D = 256
_EPS = 1e-12
_RB = 80
_BW = _RB * _D
_NW = 32

_N_SC = 12000
_TC_B = 1000
_N_TC = _N - _N_SC
_NBLK = _N_SC // _RB
_MAXITER = 2 * (-(-(-(-_NBLK // _NW)) // 2))


def _rsqrt_nr(s):
    i = lax.bitcast_convert_type(s, jnp.int32)
    i = jnp.int32(0x5F3759DF) - lax.shift_right_logical(i, 1)
    r = lax.bitcast_convert_type(i, jnp.float32)
    for _ in range(3):
        r = r * (1.5 - 0.5 * s * r * r)
    return r


def _pair_contrib(sp, sq, d):
    rp = jnp.where(sp >= _EPS * _EPS, _rsqrt_nr(sp), 1.0 / _EPS)
    rq = jnp.where(sq >= _EPS * _EPS, _rsqrt_nr(sq), 1.0 / _EPS)
    return sp * rp * rp + sq * rq * rq - 2.0 * d * rp * rq


def _diag_indices():
    lanes = lax.iota(jnp.int32, 16)
    out = []
    for k in range(16):
        col = jnp.where(lanes < 16 - k, lanes + k, lanes + k - 16)
        out.append(lanes * _D + col)
    return out


def _group_contrib(bp, bq, idxs, gbase):
    zeros = jnp.zeros((16,), jnp.float32)

    def sbody(s, carry):
        spv, sqv, dv = carry
        base = gbase + s * 16
        for k in range(16):
            off = idxs[k] + base
            vp = plsc.load_gather(bp, [off])
            vq = plsc.load_gather(bq, [off])
            spv = spv + vp * vp
            sqv = sqv + vq * vq
            dv = dv + vp * vq
        return spv, sqv, dv

    spv, sqv, dv = lax.fori_loop(0, _D // 16, sbody, (zeros,) * 3)
    return _pair_contrib(spv, sqv, dv)


def _make_sc_call():
    mesh = plsc.VectorSubcoreMesh(core_axis_name="c", subcore_axis_name="s")

    @functools.partial(
        pl.kernel,
        mesh=mesh,
        compiler_params=pltpu.CompilerParams(needs_layout_passes=False),
        out_type=jax.ShapeDtypeStruct((_NW * 16,), jnp.float32),
        scratch_types=[
            pltpu.VMEM((_BW,), jnp.float32),
            pltpu.VMEM((_BW,), jnp.float32),
            pltpu.VMEM((_BW,), jnp.float32),
            pltpu.VMEM((_BW,), jnp.float32),
            pltpu.VMEM((16,), jnp.float32),
            pltpu.SemaphoreType.DMA,
            pltpu.SemaphoreType.DMA,
        ],
    )
    def sc_call(p1h, q2h, p2h, q1h, outh,
                ap, aq, bp, bq, accv, sem_a, sem_b):
        c = lax.axis_index("c")
        s = lax.axis_index("s")
        wid = s * 2 + c
        accv[...] = jnp.zeros((16,), jnp.float32)
        idxs = _diag_indices()
        slots = ((ap, aq, sem_a), (bp, bq, sem_b))

        def phase(ph, qh):
            hbm = (ph, qh)

            def issue(i, slot):
                blk = wid + i * _NW

                @pl.when(blk < _NBLK)
                def _():
                    base = blk * _BW
                    for src, dst in zip(hbm, slot[:2]):
                        pltpu.async_copy(
                            src.at[pl.ds(base, _BW)], dst, slot[2])

            def drain_compute(i, slot):
                blk = wid + i * _NW

                @pl.when(blk < _NBLK)
                def _():
                    base = blk * _BW
                    for src, dst in zip(hbm, slot[:2]):
                        pltpu.make_async_copy(
                            src.at[pl.ds(base, _BW)], dst, slot[2]).wait()
                    for g in range(_RB // 16):
                        accv[...] = accv[...] + _group_contrib(
                            slot[0], slot[1], idxs, g * 16 * _D)

            issue(0, slots[0])

            def pair_body(i2, _):
                i = i2 * 2
                issue(i + 1, slots[1])
                drain_compute(i, slots[0])
                issue(i + 2, slots[0])
                drain_compute(i + 1, slots[1])
                return 0

            lax.fori_loop(0, _MAXITER // 2, pair_body, 0)

        phase(p1h, q2h)
        phase(p2h, q1h)
        pltpu.sync_copy(accv, outh.at[pl.ds(wid * 16, 16)])

    return sc_call


_sc_call = _make_sc_call()


def _tc_body(p1, q2, p2, q1, out):
    def pair(a, b):
        av = a[...]
        bv = b[...]
        sa = jnp.sum(av * av, axis=1, keepdims=True)
        sb = jnp.sum(bv * bv, axis=1, keepdims=True)
        d = jnp.sum(av * bv, axis=1, keepdims=True)
        na = jnp.maximum(jnp.sqrt(sa), _EPS)
        nb = jnp.maximum(jnp.sqrt(sb), _EPS)
        return sa / (na * na) + sb / (nb * nb) - 2.0 * d / (na * nb)

    c = pair(p1, q2) + pair(p2, q1)
    out[...] = jnp.full((8, 128), jnp.sum(c), jnp.float32)


_GRID_TC = _N_TC // _TC_B
_OFF_TC = _N_SC // _TC_B


def _tc_call(pred1, proj2, pred2, proj1):
    in_spec = pl.BlockSpec((_TC_B, _D), lambda i: (i + _OFF_TC, 0))
    return pl.pallas_call(
        _tc_body,
        grid=(_GRID_TC,),
        in_specs=[in_spec] * 4,
        out_specs=pl.BlockSpec((8, 128), lambda i: (i, 0)),
        out_shape=jax.ShapeDtypeStruct((_GRID_TC * 8, 128), jnp.float32),
    )(pred1, proj2, pred2, proj1)


def kernel(pred1, proj2, pred2, proj1):
    sc_partials = _sc_call(
        pred1[:_N_SC].reshape(-1), proj2[:_N_SC].reshape(-1),
        pred2[:_N_SC].reshape(-1), proj1[:_N_SC].reshape(-1))
    tc_partials = _tc_call(pred1, proj2, pred2, proj1)
    return 0.5 * (jnp.sum(sc_partials)
                  + jnp.sum(tc_partials) * (1.0 / (8.0 * 128.0)))
